# baseline (speedup 1.0000x reference)
import jax
import jax.numpy as jnp
from jax import lax
from jax.experimental import pallas as pl
from jax.experimental.pallas import tpu as pltpu
from jax.experimental.pallas import tpu_sc as plsc

_N = 201
_PAD = 416
_WIN = 224
_NCHUNK = 13


def _dilate_body(pad_hbm, scale_hbm, out_hbm, win_v, scale_v, out_v,
                 sem_w, sem_b, sem_s):
    wid = lax.axis_index("s")

    @pl.when(wid < _NCHUNK)
    def _():
        j0 = wid * 16
        cp_win_a = pltpu.async_copy(pad_hbm.at[pl.ds(j0, 128)],
                                    win_v.at[pl.ds(0, 128)], sem_w)
        cp_win_b = pltpu.async_copy(pad_hbm.at[pl.ds(j0 + 128, _WIN - 128)],
                                    win_v.at[pl.ds(128, _WIN - 128)], sem_b)
        cp_scale = pltpu.async_copy(scale_hbm, scale_v, sem_s)

        cp_scale.wait()
        neg_inv4 = jnp.float32(-0.25) / scale_v[...]
        fiota = lax.iota(jnp.int32, 16).astype(jnp.float32)
        cp_win_a.wait()

        def hchunk(base_f):
            z = (fiota + base_f) * jnp.float32(0.995) - jnp.float32(99.0)
            return (z * z) * neg_inv4

        def chunk_body(b, a):
            hv = hchunk(jnp.float32(16.0) * b.astype(jnp.float32))
            base = 16 * b
            for t in range(16):
                seg = win_v[pl.ds(base + t, 16)]
                a = jnp.maximum(a, seg + hv[t])
            return a

        acc0 = plsc.parallel_loop(
            0, 7, 1, unroll=3,
            carry=jnp.full((16,), -jnp.inf, dtype=jnp.float32))(chunk_body)
        cp_win_b.wait()
        acc = plsc.parallel_loop(7, _NCHUNK - 1, 1, unroll=3,
                                 carry=acc0)(chunk_body)
        hv = hchunk(jnp.float32(192.0))
        for t in range(_N - 192):
            seg = win_v[pl.ds(192 + t, 16)]
            acc = jnp.maximum(acc, seg + hv[t])
        out_v[...] = acc
        pltpu.sync_copy(out_v, out_hbm.at[pl.ds(j0, 16)])


_dilate = pl.kernel(
    _dilate_body,
    out_type=jax.ShapeDtypeStruct((_NCHUNK * 16,), jnp.float32),
    mesh=plsc.VectorSubcoreMesh(core_axis_name="c", subcore_axis_name="s",
                                num_cores=1),
    scratch_types=[
        pltpu.VMEM((_WIN,), jnp.float32),
        pltpu.VMEM((16,), jnp.float32),
        pltpu.VMEM((16,), jnp.float32),
        pltpu.SemaphoreType.DMA,
        pltpu.SemaphoreType.DMA,
        pltpu.SemaphoreType.DMA,
    ],
)


@jax.jit
def kernel(input, scale):
    pad = jnp.full((_PAD,), -jnp.inf, dtype=jnp.float32)
    pad = lax.dynamic_update_slice(pad, input.astype(jnp.float32), (100,))
    scale_vec = jnp.broadcast_to(scale.astype(jnp.float32), (16,))
    out = _dilate(pad, scale_vec)
    return out[:_N]


# speedup vs baseline: 1.0089x; 1.0089x over previous
import jax
import jax.numpy as jnp
from jax import lax
from jax.experimental import pallas as pl
from jax.experimental.pallas import tpu as pltpu
from jax.experimental.pallas import tpu_sc as plsc

_N = 201
_PAD = 416
_WIN = 224
_NCHUNK = 13


def _dilate_body(pad_hbm, scale_hbm, out_hbm, win_v, scale_v, out_v,
                 sem_w, sem_s):
    wid = lax.axis_index("s")

    @pl.when(wid < _NCHUNK)
    def _():
        j0 = wid * 16
        cp_win = pltpu.async_copy(pad_hbm.at[pl.ds(j0, _WIN)], win_v, sem_w)
        cp_scale = pltpu.async_copy(scale_hbm, scale_v, sem_s)

        cp_scale.wait()
        neg_inv4 = jnp.float32(-0.25) / scale_v[...]
        fiota = lax.iota(jnp.int32, 16).astype(jnp.float32)
        cp_win.wait()

        def hchunk(base_f):
            z = (fiota + base_f) * jnp.float32(0.995) - jnp.float32(99.0)
            return (z * z) * neg_inv4

        @plsc.parallel_loop(0, _NCHUNK - 1, 1, unroll=3,
                            carry=jnp.full((16,), -jnp.inf, dtype=jnp.float32))
        def acc(b, a):
            hv = hchunk(jnp.float32(16.0) * b.astype(jnp.float32))
            base = 16 * b
            for t in range(16):
                seg = win_v[pl.ds(base + t, 16)]
                a = jnp.maximum(a, seg + hv[t])
            return a
        hv = hchunk(jnp.float32(192.0))
        for t in range(_N - 192):
            seg = win_v[pl.ds(192 + t, 16)]
            acc = jnp.maximum(acc, seg + hv[t])
        out_v[...] = acc
        pltpu.sync_copy(out_v, out_hbm.at[pl.ds(j0, 16)])


_dilate = pl.kernel(
    _dilate_body,
    out_type=jax.ShapeDtypeStruct((_NCHUNK * 16,), jnp.float32),
    mesh=plsc.VectorSubcoreMesh(core_axis_name="c", subcore_axis_name="s",
                                num_cores=1),
    scratch_types=[
        pltpu.VMEM((_WIN,), jnp.float32),
        pltpu.VMEM((16,), jnp.float32),
        pltpu.VMEM((16,), jnp.float32),
        pltpu.SemaphoreType.DMA,
        pltpu.SemaphoreType.DMA,
    ],
)


@jax.jit
def kernel(input, scale):
    pad = jnp.full((_PAD,), -jnp.inf, dtype=jnp.float32)
    pad = lax.dynamic_update_slice(pad, input.astype(jnp.float32), (100,))
    scale_vec = jnp.broadcast_to(scale.astype(jnp.float32), (16,))
    out = _dilate(pad, scale_vec)
    return out[:_N]


# speedup vs baseline: 1.0236x; 1.0146x over previous
import jax
import jax.numpy as jnp
from jax import lax
from jax.experimental import pallas as pl
from jax.experimental.pallas import tpu as pltpu
from jax.experimental.pallas import tpu_sc as plsc

_N = 201
_PAD = 416
_WIN = 224
_NCHUNK = 13


def _dilate_body(pad_hbm, out_hbm, win_v, scale_v, out_v, sem_w, sem_s):
    wid = lax.axis_index("s")

    @pl.when(wid < _NCHUNK)
    def _():
        j0 = wid * 16
        cp_win = pltpu.async_copy(pad_hbm.at[pl.ds(j0, _WIN)], win_v, sem_w)
        cp_scale = pltpu.async_copy(pad_hbm.at[pl.ds(_PAD, 16)], scale_v,
                                    sem_s)

        cp_scale.wait()
        neg_inv4 = jnp.float32(-0.25) / scale_v[...]
        fiota = lax.iota(jnp.int32, 16).astype(jnp.float32)
        cp_win.wait()

        def hchunk(base_f):
            z = (fiota + base_f) * jnp.float32(0.995) - jnp.float32(99.0)
            return (z * z) * neg_inv4

        @plsc.parallel_loop(0, _NCHUNK - 1, 1, unroll=3,
                            carry=jnp.full((16,), -jnp.inf, dtype=jnp.float32))
        def acc(b, a):
            hv = hchunk(jnp.float32(16.0) * b.astype(jnp.float32))
            base = 16 * b
            for t in range(16):
                seg = win_v[pl.ds(base + t, 16)]
                a = jnp.maximum(a, seg + hv[t])
            return a
        hv = hchunk(jnp.float32(192.0))
        for t in range(_N - 192):
            seg = win_v[pl.ds(192 + t, 16)]
            acc = jnp.maximum(acc, seg + hv[t])
        out_v[...] = acc
        pltpu.sync_copy(out_v, out_hbm.at[pl.ds(j0, 16)])


_dilate = pl.kernel(
    _dilate_body,
    out_type=jax.ShapeDtypeStruct((_NCHUNK * 16,), jnp.float32),
    mesh=plsc.VectorSubcoreMesh(core_axis_name="c", subcore_axis_name="s",
                                num_cores=1),
    scratch_types=[
        pltpu.VMEM((_WIN,), jnp.float32),
        pltpu.VMEM((16,), jnp.float32),
        pltpu.VMEM((16,), jnp.float32),
        pltpu.SemaphoreType.DMA,
        pltpu.SemaphoreType.DMA,
    ],
)


@jax.jit
def kernel(input, scale):
    buf = jnp.full((_PAD + 16,), -jnp.inf, dtype=jnp.float32)
    buf = lax.dynamic_update_slice(buf, input.astype(jnp.float32), (100,))
    buf = lax.dynamic_update_slice(
        buf, jnp.broadcast_to(scale.astype(jnp.float32), (16,)), (_PAD,))
    out = _dilate(buf)
    return out[:_N]


# speedup vs baseline: 1.0244x; 1.0008x over previous
import jax
import jax.numpy as jnp
from jax import lax
from jax.experimental import pallas as pl
from jax.experimental.pallas import tpu as pltpu
from jax.experimental.pallas import tpu_sc as plsc

_N = 201
_PAD = 416
_WIN = 224
_NCHUNK = 13


def _dilate_body(pad_hbm, out_hbm, win_v, scale_v, sem_w, sem_s):
    wid = lax.axis_index("s")

    @pl.when(wid < _NCHUNK)
    def _():
        j0 = wid * 16
        cp_win = pltpu.async_copy(pad_hbm.at[pl.ds(j0, _WIN)], win_v, sem_w)
        cp_scale = pltpu.async_copy(pad_hbm.at[pl.ds(_PAD, 16)], scale_v,
                                    sem_s)

        cp_scale.wait()
        neg_inv4 = jnp.float32(-0.25) / scale_v[...]
        fiota = lax.iota(jnp.int32, 16).astype(jnp.float32)
        cp_win.wait()

        def hchunk(base_f):
            z = (fiota + base_f) * jnp.float32(0.995) - jnp.float32(99.0)
            return (z * z) * neg_inv4

        @plsc.parallel_loop(0, _NCHUNK - 1, 1, unroll=3,
                            carry=jnp.full((16,), -jnp.inf, dtype=jnp.float32))
        def acc(b, a):
            hv = hchunk(jnp.float32(16.0) * b.astype(jnp.float32))
            base = 16 * b
            for t in range(16):
                seg = win_v[pl.ds(base + t, 16)]
                a = jnp.maximum(a, seg + hv[t])
            return a
        hv = hchunk(jnp.float32(192.0))
        for t in range(_N - 192):
            seg = win_v[pl.ds(192 + t, 16)]
            acc = jnp.maximum(acc, seg + hv[t])
        scale_v[...] = acc
        pltpu.sync_copy(scale_v, out_hbm.at[pl.ds(j0, 16)])


_dilate = pl.kernel(
    _dilate_body,
    out_type=jax.ShapeDtypeStruct((_NCHUNK * 16,), jnp.float32),
    mesh=plsc.VectorSubcoreMesh(core_axis_name="c", subcore_axis_name="s",
                                num_cores=1),
    scratch_types=[
        pltpu.VMEM((_WIN,), jnp.float32),
        pltpu.VMEM((16,), jnp.float32),
        pltpu.SemaphoreType.DMA,
        pltpu.SemaphoreType.DMA,
    ],
)


@jax.jit
def kernel(input, scale):
    buf = jnp.full((_PAD + 16,), -jnp.inf, dtype=jnp.float32)
    buf = lax.dynamic_update_slice(buf, input.astype(jnp.float32), (100,))
    buf = lax.dynamic_update_slice(
        buf, jnp.broadcast_to(scale.astype(jnp.float32), (16,)), (_PAD,))
    out = _dilate(buf)
    return out[:_N]


# speedup vs baseline: 1.0253x; 1.0009x over previous
import jax
import jax.numpy as jnp
from jax import lax
from jax.experimental import pallas as pl
from jax.experimental.pallas import tpu as pltpu
from jax.experimental.pallas import tpu_sc as plsc

_N = 201
_PAD = 416
_WIN = 224
_NCHUNK = 13


def _dilate_body(pad_hbm, out_hbm, win_v, scale_v, sem_w, sem_s):
    wid = lax.axis_index("s")

    @pl.when(wid < _NCHUNK)
    def _():
        j0 = wid * 16
        cp_win = pltpu.async_copy(pad_hbm.at[pl.ds(j0, _WIN)], win_v, sem_w)
        cp_scale = pltpu.async_copy(pad_hbm.at[pl.ds(_PAD, 16)], scale_v,
                                    sem_s)

        cp_scale.wait()
        neg_inv4 = jnp.float32(-0.25) / scale_v[...]
        fiota = lax.iota(jnp.int32, 16).astype(jnp.float32)
        cp_win.wait()

        def hchunk(base_f):
            z = (fiota + base_f) * jnp.float32(0.995) - jnp.float32(99.0)
            return (z * z) * neg_inv4

        @plsc.parallel_loop(0, _NCHUNK - 1, 1, unroll=4,
                            carry=jnp.full((16,), -jnp.inf, dtype=jnp.float32))
        def acc(b, a):
            hv = hchunk(jnp.float32(16.0) * b.astype(jnp.float32))
            base = 16 * b
            for t in range(16):
                seg = win_v[pl.ds(base + t, 16)]
                a = jnp.maximum(a, seg + hv[t])
            return a
        hv = hchunk(jnp.float32(192.0))
        for t in range(_N - 192):
            seg = win_v[pl.ds(192 + t, 16)]
            acc = jnp.maximum(acc, seg + hv[t])
        scale_v[...] = acc
        pltpu.sync_copy(scale_v, out_hbm.at[pl.ds(j0, 16)])


_dilate = pl.kernel(
    _dilate_body,
    out_type=jax.ShapeDtypeStruct((_NCHUNK * 16,), jnp.float32),
    mesh=plsc.VectorSubcoreMesh(core_axis_name="c", subcore_axis_name="s",
                                num_cores=1),
    scratch_types=[
        pltpu.VMEM((_WIN,), jnp.float32),
        pltpu.VMEM((16,), jnp.float32),
        pltpu.SemaphoreType.DMA,
        pltpu.SemaphoreType.DMA,
    ],
)


@jax.jit
def kernel(input, scale):
    buf = jnp.full((_PAD + 16,), -jnp.inf, dtype=jnp.float32)
    buf = lax.dynamic_update_slice(buf, input.astype(jnp.float32), (100,))
    buf = lax.dynamic_update_slice(
        buf, jnp.broadcast_to(scale.astype(jnp.float32), (16,)), (_PAD,))
    out = _dilate(buf)
    return out[:_N]
